# Initial kernel scaffold; baseline (speedup 1.0000x reference)
import functools

import jax
import jax.numpy as jnp
from jax import lax
from jax.experimental import pallas as pl
from jax.experimental.pallas import tpu as pltpu
from jax.experimental.pallas import tpu_sc as plsc

NC, NS = 2, 16
NW = NC * NS
B, L, EMB = 16384, 20, 32
TOT = B * L
BPW = TOT // NW
C = 1024
NCHUNK = BPW // C

_mesh = plsc.VectorSubcoreMesh(
    core_axis_name="c", subcore_axis_name="s", num_cores=NC, num_subcores=NS
)


@functools.partial(
    pl.kernel,
    mesh=_mesh,
    out_type=jax.ShapeDtypeStruct((TOT, EMB), jnp.float32),
    scratch_types=[
        pltpu.VMEM((C,), jnp.int32),
        pltpu.VMEM((C, EMB), jnp.float32),
        pltpu.SemaphoreType.DMA,
    ],
)
def _gather(y_hbm, table_hbm, out_hbm, idx_v, rows_v, sem):
    wid = lax.axis_index("s") * NC + lax.axis_index("c")
    base = wid * BPW

    def step(i, carry):
        off = base + i * C
        pltpu.sync_copy(y_hbm.at[pl.ds(off, C)], idx_v)
        pltpu.async_copy(table_hbm.at[idx_v], rows_v, sem).wait()
        pltpu.sync_copy(rows_v, out_hbm.at[pl.ds(off, C)])
        return carry

    lax.fori_loop(0, NCHUNK, step, 0)


def kernel(y, table):
    out = _gather(y.reshape(TOT), table)
    return out.reshape(B, L, EMB)


# speedup vs baseline: 1.4935x; 1.4935x over previous
import functools

import jax
import jax.numpy as jnp
from jax import lax
from jax.experimental import pallas as pl
from jax.experimental.pallas import tpu as pltpu
from jax.experimental.pallas import tpu_sc as plsc

NC, NS = 2, 16
NW = NC * NS
B, L, EMB = 16384, 20, 32
TOT = B * L
BPW = TOT // NW
C = 1024
NCHUNK = BPW // C

_mesh = plsc.VectorSubcoreMesh(
    core_axis_name="c", subcore_axis_name="s", num_cores=NC, num_subcores=NS
)


@functools.partial(
    pl.kernel,
    mesh=_mesh,
    out_type=jax.ShapeDtypeStruct((TOT, EMB), jnp.float32),
    scratch_types=[
        pltpu.VMEM((C,), jnp.int32),
        pltpu.VMEM((C, EMB), jnp.float32),
        pltpu.SemaphoreType.DMA,
    ],
    compiler_params=pltpu.CompilerParams(use_tc_tiling_on_sc=False),
)
def _gather(y_hbm, table_hbm, out_hbm, idx_v, rows_v, sem):
    wid = lax.axis_index("s") * NC + lax.axis_index("c")
    base = wid * BPW

    def step(i, carry):
        off = base + i * C
        pltpu.sync_copy(y_hbm.at[pl.ds(off, C)], idx_v)
        pltpu.async_copy(table_hbm.at[idx_v], rows_v, sem).wait()
        pltpu.sync_copy(rows_v, out_hbm.at[pl.ds(off, C)])
        return carry

    lax.fori_loop(0, NCHUNK, step, 0)


def kernel(y, table):
    out = _gather(y.reshape(TOT), table)
    return out.reshape(B, L, EMB)


# speedup vs baseline: 1.5151x; 1.0144x over previous
import functools

import jax
import jax.numpy as jnp
from jax import lax
from jax.experimental import pallas as pl
from jax.experimental.pallas import tpu as pltpu
from jax.experimental.pallas import tpu_sc as plsc

NC, NS = 2, 16
NW = NC * NS
B, L, EMB = 16384, 20, 32
TOT = B * L
BPW = TOT // NW
C = 1024
NCHUNK = BPW // C
NBUF = 3

_mesh = plsc.VectorSubcoreMesh(
    core_axis_name="c", subcore_axis_name="s", num_cores=NC, num_subcores=NS
)


@functools.partial(
    pl.kernel,
    mesh=_mesh,
    out_type=jax.ShapeDtypeStruct((TOT, EMB), jnp.float32),
    scratch_types=[
        pltpu.VMEM((BPW,), jnp.int32),
        pltpu.VMEM((NBUF, C, EMB), jnp.float32),
        pltpu.SemaphoreType.DMA((NBUF,)),
        pltpu.SemaphoreType.DMA((NBUF,)),
    ],
    compiler_params=pltpu.CompilerParams(use_tc_tiling_on_sc=False),
)
def _gather(y_hbm, table_hbm, out_hbm, idx_v, rows_v, gsem, ssem):
    wid = lax.axis_index("s") * NC + lax.axis_index("c")
    base = wid * BPW

    pltpu.sync_copy(y_hbm.at[pl.ds(base, BPW)], idx_v)

    def fire_gather(c):
        return pltpu.async_copy(
            table_hbm.at[idx_v.at[pl.ds(c * C, C)]],
            rows_v.at[c % NBUF],
            gsem.at[c % NBUF],
        )

    def fire_store(c):
        return pltpu.async_copy(
            rows_v.at[c % NBUF],
            out_hbm.at[pl.ds(base + c * C, C)],
            ssem.at[c % NBUF],
        )

    gathers = {}
    stores = {}
    for c in range(min(2, NCHUNK)):
        gathers[c] = fire_gather(c)
    for c in range(NCHUNK):
        nxt = c + 2
        if nxt < NCHUNK:
            prev = nxt - NBUF
            if prev >= 0:
                stores.pop(prev).wait()
            gathers[nxt] = fire_gather(nxt)
        gathers.pop(c).wait()
        stores[c] = fire_store(c)
    for c in sorted(stores):
        stores.pop(c).wait()


def kernel(y, table):
    out = _gather(y.reshape(TOT), table)
    return out.reshape(B, L, EMB)
